# baseline (speedup 1.0000x reference)
import functools

import jax
import jax.numpy as jnp
from jax import lax
from jax.experimental import pallas as pl
from jax.experimental.pallas import tpu as pltpu
from jax.experimental.pallas import tpu_sc as plsc

B, L, D = 4096, 50, 64
N = B * L
NC, NS = 2, 16
NW = NC * NS
C = 128
RPW = N // NW
G = RPW // C
NB = 8

_mesh = plsc.VectorSubcoreMesh(core_axis_name="c", subcore_axis_name="s")


@functools.partial(
    pl.kernel,
    mesh=_mesh,
    out_type=jax.ShapeDtypeStruct((N, D), jnp.float32),
    scratch_types=[
        pltpu.VMEM((G, C), jnp.int32),
        pltpu.VMEM((NB, C, D), jnp.float32),
        pltpu.SemaphoreType.DMA,
        pltpu.SemaphoreType.DMA,
    ],
    compiler_params=pltpu.CompilerParams(use_tc_tiling_on_sc=False),
)
def _embed_sc(idx_hbm, table_hbm, out_hbm, idx_v, rows_v, gsem, ssem):
    wid = lax.axis_index("s") * NC + lax.axis_index("c")
    base = wid * RPW
    pltpu.sync_copy(idx_hbm.at[wid], idx_v)

    def gather(g, b):
        return pltpu.make_async_copy(
            table_hbm.at[idx_v.at[g]], rows_v.at[b], gsem)

    def store(g, b):
        return pltpu.make_async_copy(
            rows_v.at[b], out_hbm.at[pl.ds(base + g * C, C)], ssem)

    for b in range(NB):
        gather(b, b).start()

    def body(g, carry):
        b = lax.rem(g, NB)
        gather(g, b).wait()
        store(g, b).start()
        g2 = g + NB - 1
        b2 = lax.rem(g2, NB)

        @pl.when(jnp.logical_and(g2 >= NB, g2 < G))
        def _():
            store(g2 - NB, b2).wait()
            gather(g2, b2).start()

        return carry

    lax.fori_loop(0, G, body, 0)
    for b in range(NB):
        store(G - NB + b, (G - NB + b) % NB).wait()


def kernel(inputs, table):
    idx = inputs.reshape(NW, G, C)
    return _embed_sc(idx, table).reshape(B, L, D)


# speedup vs baseline: 1.2768x; 1.2768x over previous
import functools

import jax
import jax.numpy as jnp
from jax import lax
from jax.experimental import pallas as pl
from jax.experimental.pallas import tpu as pltpu
from jax.experimental.pallas import tpu_sc as plsc

B, L, D = 4096, 50, 64
LP = 56
NC, NS = 2, 16
NW = NC * NS
C = 128
NB = 8

_mesh = plsc.VectorSubcoreMesh(core_axis_name="c", subcore_axis_name="s")


@functools.partial(
    pl.kernel,
    mesh=_mesh,
    out_type=jax.ShapeDtypeStruct((B, LP, 128), jnp.float32),
    scratch_types=[
        pltpu.VMEM((L, C), jnp.int32),
        pltpu.VMEM((NB, C, D), jnp.float32),
        pltpu.SemaphoreType.DMA,
        pltpu.SemaphoreType.DMA,
    ],
    compiler_params=pltpu.CompilerParams(use_tc_tiling_on_sc=False),
)
def _embed_sc(idx_hbm, table_hbm, out_hbm, idx_v, rows_v, gsem, ssem):
    wid = lax.axis_index("s") * NC + lax.axis_index("c")
    b0 = wid * C
    pltpu.sync_copy(idx_hbm.at[wid], idx_v)

    def gather(l, slot):
        return pltpu.make_async_copy(
            table_hbm.at[idx_v.at[l]], rows_v.at[slot], gsem)

    def store(l, slot):
        return pltpu.make_async_copy(
            rows_v.at[slot], out_hbm.at[pl.ds(b0, C), l, pl.ds(0, D)], ssem)

    for slot in range(NB):
        gather(slot, slot).start()

    def body(l, carry):
        slot = lax.rem(l, NB)
        gather(l, slot).wait()
        store(l, slot).start()
        l2 = l + NB - 1
        s2 = lax.rem(l2, NB)

        @pl.when(jnp.logical_and(l2 >= NB, l2 < L))
        def _():
            store(l2 - NB, s2).wait()
            gather(l2, s2).start()

        return carry

    lax.fori_loop(0, L, body, 0)
    for i in range(NB):
        store(L - NB + i, (L - NB + i) % NB).wait()


def _tpose_body(x_ref, o_ref):
    for l in range(L):
        x = x_ref[:, l, :]
        t = jnp.transpose(x)
        o_ref[l, :, 0] = t[0:D].reshape(8, 8, C)


_tpose = pl.pallas_call(
    _tpose_body,
    grid=(NW,),
    in_specs=[pl.BlockSpec((C, LP, 128), lambda w: (w, 0, 0))],
    out_specs=pl.BlockSpec((L, 8, 1, 8, C), lambda w: (0, 0, w, 0, 0)),
    out_shape=jax.ShapeDtypeStruct((L, 8, NW, 8, C), jnp.float32),
    compiler_params=pltpu.CompilerParams(dimension_semantics=("parallel",)),
)


def kernel(inputs, table):
    idxw = inputs.T.reshape(L, NW, C).transpose(1, 0, 2)
    pad = _embed_sc(idxw, table)
    out5 = _tpose(pad)
    return out5.transpose(2, 4, 0, 1, 3).reshape(B, L, D)


# speedup vs baseline: 1.5501x; 1.2140x over previous
import functools

import jax
import jax.numpy as jnp
from jax import lax
from jax.experimental import pallas as pl
from jax.experimental.pallas import tpu as pltpu
from jax.experimental.pallas import tpu_sc as plsc

B, L, D = 4096, 50, 64
LH = L // 2
NC, NS = 2, 16
NW = NC * NS
C = 128
NB = 8

_mesh = plsc.VectorSubcoreMesh(core_axis_name="c", subcore_axis_name="s")


@functools.partial(
    pl.kernel,
    mesh=_mesh,
    out_type=jax.ShapeDtypeStruct((LH, NW, C, 128), jnp.float32),
    scratch_types=[
        pltpu.VMEM((L, C), jnp.int32),
        pltpu.VMEM((NB, C, D), jnp.float32),
        pltpu.SemaphoreType.DMA,
        pltpu.SemaphoreType.DMA,
    ],
    compiler_params=pltpu.CompilerParams(use_tc_tiling_on_sc=False),
)
def _embed_sc(idx_hbm, table_hbm, out_hbm, idx_v, rows_v, gsem, ssem):
    wid = lax.axis_index("s") * NC + lax.axis_index("c")
    pltpu.sync_copy(idx_hbm.at[wid], idx_v)

    def gather(l, slot):
        return pltpu.make_async_copy(
            table_hbm.at[idx_v.at[l]], rows_v.at[slot], gsem)

    def store(l, slot):
        lp = lax.shift_right_logical(l, 1)
        ho = lax.rem(l, 2) * D
        return pltpu.make_async_copy(
            rows_v.at[slot], out_hbm.at[lp, wid, :, pl.ds(ho, D)], ssem)

    for slot in range(NB):
        gather(slot, slot).start()

    def body(l, carry):
        slot = lax.rem(l, NB)
        gather(l, slot).wait()
        store(l, slot).start()
        l2 = l + NB - 1
        s2 = lax.rem(l2, NB)

        @pl.when(jnp.logical_and(l2 >= NB, l2 < L))
        def _():
            store(l2 - NB, s2).wait()
            gather(l2, s2).start()

        return carry

    lax.fori_loop(0, L, body, 0)
    for i in range(NB):
        store(L - NB + i, (L - NB + i) % NB).wait()


def _tpose_body(x_ref, o_ref):
    for bb in range(NW):
        x = x_ref[0, bb]
        t = jnp.transpose(x)
        o_ref[0, :, bb] = t[0:D].reshape(8, 8, C)
        o_ref[1, :, bb] = t[D:2 * D].reshape(8, 8, C)


_tpose = pl.pallas_call(
    _tpose_body,
    grid=(LH,),
    in_specs=[pl.BlockSpec((1, NW, C, 128), lambda lp: (lp, 0, 0, 0))],
    out_specs=pl.BlockSpec((2, 8, NW, 8, C), lambda lp: (lp, 0, 0, 0, 0)),
    out_shape=jax.ShapeDtypeStruct((L, 8, NW, 8, C), jnp.float32),
    compiler_params=pltpu.CompilerParams(dimension_semantics=("parallel",)),
)


def kernel(inputs, table):
    idxw = inputs.T.reshape(L, NW, C).transpose(1, 0, 2)
    packed = _embed_sc(idxw, table)
    out5 = _tpose(packed)
    return out5.transpose(2, 4, 0, 1, 3).reshape(B, L, D)
